# baseline (speedup 1.0000x reference)
import functools

import jax
import jax.numpy as jnp
from jax import lax
from jax.experimental import pallas as pl
from jax.experimental.pallas import tpu as pltpu
from jax.experimental.pallas import tpu_sc as plsc

L = 16
NC = 2
NS = 16
NW = NC * NS
NPHASE = 2
NPW = 160
ECH = 2048
GB = 64
SELCAP = 2560
SENTINEL = 1 << 30


def _leaky(v):
    return jnp.where(v >= 0, v, 0.01 * v)


@functools.lru_cache(maxsize=None)
def _build(N, E, D):
    NPAD = NPHASE * NW * NPW
    assert NPAD >= N
    NCH = -(-E // ECH)
    EP = NCH * ECH
    DC = D // L

    mesh = plsc.VectorSubcoreMesh(core_axis_name="c", subcore_axis_name="s",
                                  num_cores=NC, num_subcores=NS)

    CAPR = SELCAP // L
    MAXR = CAPR - ECH // L - 8

    @functools.partial(
        pl.kernel,
        out_type=[jax.ShapeDtypeStruct((NPAD * D,), jnp.float32)] * 3,
        mesh=mesh,
        scratch_types=[
            pltpu.VMEM(((NPW + 1) * D,), jnp.float32),
            pltpu.VMEM(((NPW + 1) * D,), jnp.float32),
            pltpu.VMEM((NPW + L,), jnp.float32),
            pltpu.VMEM((2 * ECH,), jnp.int32),
            pltpu.VMEM((2 * ECH,), jnp.int32),
            pltpu.VMEM((SELCAP + L,), jnp.int32),
            pltpu.VMEM((SELCAP + L,), jnp.int32),
            pltpu.VMEM((2 * GB, D), jnp.float32),
            pltpu.SemaphoreType.DMA,
            pltpu.SemaphoreType.DMA,
        ],
        compiler_params=pltpu.CompilerParams(use_tc_tiling_on_sc=False,
                                             needs_layout_passes=False),
    )
    def sc_seg(y_h, src_h, dst_h, sum_h, mean_h, max_h,
               acc_sum, acc_max, acc_cnt, csrc, cdst, ssrc, sdst, gbuf,
               sem_c, sem_g):
        cid = lax.axis_index("c")
        sid = lax.axis_index("s")
        wid = sid * NC + cid
        lanes = lax.iota(jnp.int32, L)
        zeros16 = jnp.zeros((L,), jnp.float32)
        ones16 = jnp.ones((L,), jnp.float32)
        snt16 = jnp.full((L,), NPW, jnp.int32)
        lane0 = lanes == 0

        def _zsel(i, _):
            v = jnp.minimum(jnp.full((L,), i * L, jnp.int32) + lanes, N - 1)
            ssrc[pl.ds(i * L, L)] = v
            sdst[pl.ds(i * L, L)] = snt16
            return 0
        lax.fori_loop(0, (SELCAP + L) // L, _zsel, 0)

        def _issue(start):
            parv = (start // GB) % 2
            off = pl.multiple_of(start % SELCAP, GB)
            idx = ssrc.at[pl.ds(off, GB)]
            pltpu.async_copy(y_h.at[idx], gbuf.at[pl.ds(parv * GB, GB)],
                             sem_g)

        def _wait_gather():
            pltpu.make_async_copy(
                y_h.at[ssrc.at[pl.ds(0, GB)]],
                gbuf.at[pl.ds(0, GB)], sem_g).wait()

        def _lanemin(v):
            for sh in (1, 2, 4, 8):
                v = jnp.minimum(v, jnp.take(v, (lanes + sh) % L))
            return v

        def _lanemax(v):
            for sh in (1, 2, 4, 8):
                v = jnp.maximum(v, jnp.take(v, (lanes + sh) % L))
            return v

        def _accum(start):
            par = (start // GB) % 2
            base = pl.multiple_of(start % SELCAP, GB)
            startv = jnp.full((L,), base, jnp.int32)

            UNR = 2

            def _edgeu(i, _):
                e0 = i * UNR
                ev = jnp.full((L,), e0, jnp.int32)
                dls = [plsc.load_gather(sdst, [startv + ev + u])
                       for u in range(UNR)]
                abase = [d[0] * D for d in dls]
                row0 = par * GB + e0
                for c in range(DC):
                    for u in range(UNR):
                        off = abase[u] + c * L
                        rv = gbuf[row0 + u, pl.ds(c * L, L)]
                        plsc.addupdate(acc_sum.at[pl.ds(off, L)], rv)
                        acc_max[pl.ds(off, L)] = jnp.maximum(
                            acc_max[pl.ds(off, L)], rv)
                for u in range(UNR):
                    ci = jnp.where(lane0, dls[u], NPW + lanes)
                    plsc.addupdate_scatter(acc_cnt, [ci], ones16)
                return 0
            lax.fori_loop(0, GB // UNR, _edgeu, 0)
            for r in range(GB // L):
                sdst[pl.ds(base + r * L, L)] = snt16

        def _drain(nb, done, issued):
            nb_end = done + nb * GB
            fresh = (nb > 0) & (issued == done)

            @pl.when(fresh)
            def _():
                _issue(done)
            issued = jnp.where(fresh, done + GB, issued)

            def _b(i, st2):
                done_i, issued_i = st2
                _wait_gather()
                has_nxt = issued_i < nb_end

                @pl.when(has_nxt)
                def _():
                    _issue(issued_i)
                issued2 = jnp.where(has_nxt, issued_i + GB, issued_i)
                _accum(done_i)
                return (done_i + GB, issued2)
            return lax.fori_loop(0, nb, _b, (done, issued))

        for phase in range(NPHASE):
            w = wid + phase * NW
            lo = w * NPW
            lov = jnp.full((L,), lo, jnp.int32)
            hiv = lov + NPW

            def _zacc(i, _):
                acc_sum[pl.ds(i * L, L)] = zeros16
                acc_max[pl.ds(i * L, L)] = zeros16
                return 0
            lax.fori_loop(0, (NPW + 1) * DC, _zacc, 0)
            for i in range((NPW + L) // L):
                acc_cnt[pl.ds(i * L, L)] = zeros16

            def _zdst(i, _):
                sdst[pl.ds(i * L, L)] = snt16
                return 0
            lax.fori_loop(0, SELCAP // L, _zdst, 0)

            pltpu.async_copy(src_h.at[pl.ds(0, ECH)],
                             csrc.at[pl.ds(0, ECH)], sem_c)
            pltpu.async_copy(dst_h.at[pl.ds(0, ECH)],
                             cdst.at[pl.ds(0, ECH)], sem_c)

            def _chunk(k, st):
                cntv, done, issued = st
                cbase = (k % 2) * ECH
                pltpu.make_async_copy(src_h.at[pl.ds(0, ECH)],
                                      csrc.at[pl.ds(0, ECH)], sem_c).wait()
                pltpu.make_async_copy(src_h.at[pl.ds(0, ECH)],
                                      cdst.at[pl.ds(0, ECH)], sem_c).wait()

                @pl.when(k + 1 < NCH)
                def _():
                    nbase = ((k + 1) % 2) * ECH
                    pltpu.async_copy(src_h.at[pl.ds((k + 1) * ECH, ECH)],
                                     csrc.at[pl.ds(nbase, ECH)], sem_c)
                    pltpu.async_copy(dst_h.at[pl.ds((k + 1) * ECH, ECH)],
                                     cdst.at[pl.ds(nbase, ECH)], sem_c)

                def _filt1(cv, cw, off):
                    d = cdst[pl.ds(off, L)]
                    s = csrc[pl.ds(off, L)]
                    m = (d >= lov) & (d < hiv)
                    pos = jnp.where(m, cw * L + lanes, SELCAP + lanes)
                    plsc.store_scatter(ssrc, [pos], s)
                    plsc.store_scatter(sdst, [pos], d - lov)
                    mi = jnp.where(m, 1, 0)
                    cw = cw + mi
                    cw = jnp.where(cw >= CAPR, cw - CAPR, cw)
                    return cv + mi, cw

                def _filt(i, st):
                    cv, cw = st
                    for u in range(2):
                        cv, cw = _filt1(cv, cw, cbase + (i * 2 + u) * L)
                    return cv, cw
                cwv = cntv - (cntv // CAPR) * CAPR
                cntv, _ = lax.fori_loop(0, ECH // (2 * L), _filt,
                                        (cntv, cwv))

                dmin = _lanemin(cntv)[0]
                dmax = _lanemax(cntv)[0]
                nb = jnp.maximum(
                    (dmin * L - done) // GB,
                    ((dmax - MAXR) * L - done + GB - 1) // GB)
                nb = jnp.maximum(nb, 0)
                done, issued = _drain(nb, done, issued)
                cntv = jnp.maximum(cntv, jnp.full((L,), done // L,
                                                  jnp.int32))

                dmin2 = _lanemin(cntv)[0]
                can = (dmin2 * L - done >= GB) & (issued == done)

                @pl.when(can)
                def _():
                    _issue(done)
                issued = jnp.where(can, done + GB, issued)
                return (cntv, done, issued)
            cntv, done, issued = lax.fori_loop(
                0, NCH, _chunk,
                (jnp.zeros((L,), jnp.int32), jnp.int32(0), jnp.int32(0)))

            dmax = _lanemax(cntv)[0]
            nb = (dmax * L - done + GB - 1) // GB
            done, issued = _drain(nb, done, issued)

            pltpu.sync_copy(acc_sum.at[pl.ds(0, NPW * D)],
                            sum_h.at[pl.ds(lo * D, NPW * D)])

            def _mean(n, _):
                nv = jnp.full((L,), n, jnp.int32)
                cv = plsc.load_gather(acc_cnt, [nv])
                r = 1.0 / jnp.maximum(cv, 1.0)
                for c in range(DC):
                    off = n * D + c * L
                    acc_sum[pl.ds(off, L)] = acc_sum[pl.ds(off, L)] * r
                return 0
            lax.fori_loop(0, NPW, _mean, 0)
            pltpu.sync_copy(acc_sum.at[pl.ds(0, NPW * D)],
                            mean_h.at[pl.ds(lo * D, NPW * D)])
            pltpu.sync_copy(acc_max.at[pl.ds(0, NPW * D)],
                            max_h.at[pl.ds(lo * D, NPW * D)])

    RB1 = 1000

    def _lin_body(x_ref, w_ref, b_ref, o_ref):
        o_ref[...] = jnp.dot(x_ref[...], w_ref[...],
                             preferred_element_type=jnp.float32) + b_ref[...]

    tc_lin = pl.pallas_call(
        _lin_body,
        grid=(N // RB1,),
        in_specs=[
            pl.BlockSpec((RB1, D), lambda i: (i, 0)),
            pl.BlockSpec((D, D), lambda i: (0, 0)),
            pl.BlockSpec((1, D), lambda i: (0, 0)),
        ],
        out_specs=pl.BlockSpec((RB1, D), lambda i: (i, 0)),
        out_shape=jax.ShapeDtypeStruct((N, D), jnp.float32),
    )

    RB2 = 1000

    def _post_body(s_ref, m_ref, x_ref, xin_ref, wa_ref, ba_ref, wg_ref,
                   bg_ref, ws_ref, bs_ref, o_ref):
        s = s_ref[...]
        mn = m_ref[...]
        mx = x_ref[...]
        xb = xin_ref[...]
        mask = (jnp.sum(jnp.abs(s), axis=1) + jnp.sum(jnp.abs(mn), axis=1)
                + jnp.sum(jnp.abs(mx), axis=1)) == 0.0
        wa = wa_ref[...]
        a = (jnp.dot(s, wa[0:D], preferred_element_type=jnp.float32)
             + jnp.dot(mn, wa[D:2 * D], preferred_element_type=jnp.float32)
             + jnp.dot(mx, wa[2 * D:3 * D], preferred_element_type=jnp.float32)
             + ba_ref[...])
        a = 1.0 / (1.0 + jnp.exp(-a))
        wg = wg_ref[...]
        t = (jnp.dot(s * a[:, 0:D], wg[0:D],
                     preferred_element_type=jnp.float32)
             + jnp.dot(mn * a[:, D:2 * D], wg[D:2 * D],
                       preferred_element_type=jnp.float32)
             + jnp.dot(mx * a[:, 2 * D:3 * D], wg[2 * D:3 * D],
                       preferred_element_type=jnp.float32)
             + bg_ref[...])
        t = jnp.where(mask[:, None], 0.0, t)
        out = _leaky(t)
        mask2 = jnp.sum(jnp.abs(out), axis=1) == 0.0
        ws = ws_ref[...]
        h = (jnp.dot(out, ws[0:D], preferred_element_type=jnp.float32)
             + jnp.dot(xb, ws[D:2 * D], preferred_element_type=jnp.float32)
             + bs_ref[...])
        h = _leaky(h)
        o_ref[...] = jnp.where(mask2[:, None], xb, h)

    tc_post = pl.pallas_call(
        _post_body,
        grid=(N // RB2,),
        in_specs=[
            pl.BlockSpec((RB2, D), lambda i: (i, 0)),
            pl.BlockSpec((RB2, D), lambda i: (i, 0)),
            pl.BlockSpec((RB2, D), lambda i: (i, 0)),
            pl.BlockSpec((RB2, D), lambda i: (i, 0)),
            pl.BlockSpec((3 * D, 3 * D), lambda i: (0, 0)),
            pl.BlockSpec((1, 3 * D), lambda i: (0, 0)),
            pl.BlockSpec((3 * D, D), lambda i: (0, 0)),
            pl.BlockSpec((1, D), lambda i: (0, 0)),
            pl.BlockSpec((2 * D, D), lambda i: (0, 0)),
            pl.BlockSpec((1, D), lambda i: (0, 0)),
        ],
        out_specs=pl.BlockSpec((RB2, D), lambda i: (i, 0)),
        out_shape=jax.ShapeDtypeStruct((N, D), jnp.float32),
    )

    def run(x, edge_index, W_msg, b_msg, W_att, b_att, W_aggr, b_aggr,
            W_self, b_self):
        y = tc_lin(x, W_msg, b_msg.reshape(1, D))
        src = edge_index[0]
        dst = edge_index[1]
        pad = EP - E
        src_p = jnp.concatenate([src, jnp.zeros((pad,), jnp.int32)])
        dst_p = jnp.concatenate([dst, jnp.full((pad,), SENTINEL, jnp.int32)])
        sum_t, mean_t, max_t = sc_seg(y, src_p, dst_p)
        sum_t = sum_t.reshape(NPAD, D)
        mean_t = mean_t.reshape(NPAD, D)
        max_t = max_t.reshape(NPAD, D)
        return tc_post(sum_t[:N], mean_t[:N], max_t[:N], x,
                       W_att, b_att.reshape(1, 3 * D),
                       W_aggr, b_aggr.reshape(1, D),
                       W_self, b_self.reshape(1, D))

    return run


def kernel(x, edge_index, W_msg, b_msg, W_att, b_att, W_aggr, b_aggr,
           W_self, b_self):
    N, D = x.shape
    E = edge_index.shape[1]
    return _build(N, E, D)(x, edge_index, W_msg, b_msg, W_att, b_att,
                           W_aggr, b_aggr, W_self, b_self)


# speedup vs baseline: 1.1265x; 1.1265x over previous
import functools

import jax
import jax.numpy as jnp
from jax import lax
from jax.experimental import pallas as pl
from jax.experimental.pallas import tpu as pltpu
from jax.experimental.pallas import tpu_sc as plsc

L = 16
NC = 2
NS = 16
NW = NC * NS
NPHASE = 2
NPW = 160
ECH = 2048
GB = 64
SELCAP = 2560
SENTINEL = 1 << 30


def _leaky(v):
    return jnp.where(v >= 0, v, 0.01 * v)


@functools.lru_cache(maxsize=None)
def _build(N, E, D):
    NPAD = NPHASE * NW * NPW
    assert NPAD >= N
    NCH = -(-E // ECH)
    EP = NCH * ECH
    DC = D // L

    mesh = plsc.VectorSubcoreMesh(core_axis_name="c", subcore_axis_name="s",
                                  num_cores=NC, num_subcores=NS)

    CAPR = SELCAP // L
    MAXR = CAPR - ECH // L - 8

    @functools.partial(
        pl.kernel,
        out_type=[jax.ShapeDtypeStruct((NPAD * D,), jnp.float32)] * 3,
        mesh=mesh,
        scratch_types=[
            pltpu.VMEM(((NPW + 1) * D,), jnp.float32),
            pltpu.VMEM(((NPW + 1) * D,), jnp.float32),
            pltpu.VMEM((NPW + L,), jnp.float32),
            pltpu.VMEM((2 * ECH,), jnp.int32),
            pltpu.VMEM((2 * ECH,), jnp.int32),
            pltpu.VMEM((SELCAP + L,), jnp.int32),
            pltpu.VMEM((SELCAP + L,), jnp.int32),
            pltpu.VMEM((2 * GB, D), jnp.float32),
            pltpu.SemaphoreType.DMA,
            pltpu.SemaphoreType.DMA,
        ],
        compiler_params=pltpu.CompilerParams(use_tc_tiling_on_sc=False,
                                             needs_layout_passes=False),
    )
    def sc_seg(y_h, src_h, dst_h, sum_h, mean_h, max_h,
               acc_sum, acc_max, acc_cnt, csrc, cdst, ssrc, sdst, gbuf,
               sem_c, sem_g):
        cid = lax.axis_index("c")
        sid = lax.axis_index("s")
        wid = sid * NC + cid
        lanes = lax.iota(jnp.int32, L)
        zeros16 = jnp.zeros((L,), jnp.float32)
        ones16 = jnp.ones((L,), jnp.float32)
        snt16 = jnp.full((L,), NPW, jnp.int32)
        lane0 = lanes == 0

        def _zsel(i, _):
            v = jnp.minimum(jnp.full((L,), i * L, jnp.int32) + lanes, N - 1)
            ssrc[pl.ds(i * L, L)] = v
            sdst[pl.ds(i * L, L)] = snt16
            return 0
        lax.fori_loop(0, (SELCAP + L) // L, _zsel, 0)

        def _issue(start):
            parv = (start // GB) % 2
            off = pl.multiple_of(start % SELCAP, GB)
            idx = ssrc.at[pl.ds(off, GB)]
            pltpu.async_copy(y_h.at[idx], gbuf.at[pl.ds(parv * GB, GB)],
                             sem_g)

        def _wait_gather():
            pltpu.make_async_copy(
                y_h.at[ssrc.at[pl.ds(0, GB)]],
                gbuf.at[pl.ds(0, GB)], sem_g).wait()

        def _lanemin(v):
            for sh in (1, 2, 4, 8):
                v = jnp.minimum(v, jnp.take(v, (lanes + sh) % L))
            return v

        def _lanemax(v):
            for sh in (1, 2, 4, 8):
                v = jnp.maximum(v, jnp.take(v, (lanes + sh) % L))
            return v

        def _accum(start):
            par = (start // GB) % 2
            base = pl.multiple_of(start % SELCAP, GB)
            startv = jnp.full((L,), base, jnp.int32)

            def _edge2(i, _):
                e0 = i * 2
                ev = jnp.full((L,), e0, jnp.int32)
                dls0 = plsc.load_gather(sdst, [startv + ev])
                dls1 = plsc.load_gather(sdst, [startv + ev + 1])
                abase0 = dls0[0] * D
                abase1 = dls1[0] * D
                row0 = par * GB + e0
                for c in range(DC):
                    off0 = abase0 + c * L
                    off1 = abase1 + c * L
                    rv0 = gbuf[row0, pl.ds(c * L, L)]
                    rv1 = gbuf[row0 + 1, pl.ds(c * L, L)]
                    plsc.addupdate(acc_sum.at[pl.ds(off0, L)], rv0)
                    acc_max[pl.ds(off0, L)] = jnp.maximum(
                        acc_max[pl.ds(off0, L)], rv0)
                    plsc.addupdate(acc_sum.at[pl.ds(off1, L)], rv1)
                    acc_max[pl.ds(off1, L)] = jnp.maximum(
                        acc_max[pl.ds(off1, L)], rv1)
                ci0 = jnp.where(lane0, dls0, NPW + lanes)
                plsc.addupdate_scatter(acc_cnt, [ci0], ones16)
                ci1 = jnp.where(lane0, dls1, NPW + lanes)
                plsc.addupdate_scatter(acc_cnt, [ci1], ones16)
                return 0
            lax.fori_loop(0, GB // 2, _edge2, 0)
            for r in range(GB // L):
                sdst[pl.ds(base + r * L, L)] = snt16

        def _drain(nb, done, issued):
            nb_end = done + nb * GB
            fresh = (nb > 0) & (issued == done)

            @pl.when(fresh)
            def _():
                _issue(done)
            issued = jnp.where(fresh, done + GB, issued)

            def _b(i, st2):
                done_i, issued_i = st2
                _wait_gather()
                has_nxt = issued_i < nb_end

                @pl.when(has_nxt)
                def _():
                    _issue(issued_i)
                issued2 = jnp.where(has_nxt, issued_i + GB, issued_i)
                _accum(done_i)
                return (done_i + GB, issued2)
            return lax.fori_loop(0, nb, _b, (done, issued))

        for phase in range(NPHASE):
            w = wid + phase * NW
            lo = w * NPW
            lov = jnp.full((L,), lo, jnp.int32)
            hiv = lov + NPW

            def _zacc(i, _):
                acc_sum[pl.ds(i * L, L)] = zeros16
                acc_max[pl.ds(i * L, L)] = zeros16
                return 0
            lax.fori_loop(0, (NPW + 1) * DC, _zacc, 0)
            for i in range((NPW + L) // L):
                acc_cnt[pl.ds(i * L, L)] = zeros16

            def _zdst(i, _):
                sdst[pl.ds(i * L, L)] = snt16
                return 0
            lax.fori_loop(0, SELCAP // L, _zdst, 0)

            pltpu.async_copy(src_h.at[pl.ds(0, ECH)],
                             csrc.at[pl.ds(0, ECH)], sem_c)
            pltpu.async_copy(dst_h.at[pl.ds(0, ECH)],
                             cdst.at[pl.ds(0, ECH)], sem_c)

            def _chunk(k, st):
                cntv, done, issued = st
                cbase = (k % 2) * ECH
                pltpu.make_async_copy(src_h.at[pl.ds(0, ECH)],
                                      csrc.at[pl.ds(0, ECH)], sem_c).wait()
                pltpu.make_async_copy(src_h.at[pl.ds(0, ECH)],
                                      cdst.at[pl.ds(0, ECH)], sem_c).wait()

                @pl.when(k + 1 < NCH)
                def _():
                    nbase = ((k + 1) % 2) * ECH
                    pltpu.async_copy(src_h.at[pl.ds((k + 1) * ECH, ECH)],
                                     csrc.at[pl.ds(nbase, ECH)], sem_c)
                    pltpu.async_copy(dst_h.at[pl.ds((k + 1) * ECH, ECH)],
                                     cdst.at[pl.ds(nbase, ECH)], sem_c)

                def _filt1(cv, cw, off):
                    d = cdst[pl.ds(off, L)]
                    s = csrc[pl.ds(off, L)]
                    m = (d >= lov) & (d < hiv)
                    pos = jnp.where(m, cw * L + lanes, SELCAP + lanes)
                    plsc.store_scatter(ssrc, [pos], s)
                    plsc.store_scatter(sdst, [pos], d - lov)
                    mi = jnp.where(m, 1, 0)
                    cw = cw + mi
                    cw = jnp.where(cw >= CAPR, cw - CAPR, cw)
                    return cv + mi, cw

                def _filt(i, st):
                    cv, cw = st
                    cv, cw = _filt1(cv, cw, cbase + i * 2 * L)
                    cv, cw = _filt1(cv, cw, cbase + i * 2 * L + L)
                    return cv, cw
                cwv = cntv - (cntv // CAPR) * CAPR
                cntv, _ = lax.fori_loop(0, ECH // (2 * L), _filt,
                                        (cntv, cwv))

                dmin = _lanemin(cntv)[0]
                dmax = _lanemax(cntv)[0]
                nb = jnp.maximum(
                    (dmin * L - done) // GB,
                    ((dmax - MAXR) * L - done + GB - 1) // GB)
                nb = jnp.maximum(nb, 0)
                done, issued = _drain(nb, done, issued)
                cntv = jnp.maximum(cntv, jnp.full((L,), done // L,
                                                  jnp.int32))

                dmin2 = _lanemin(cntv)[0]
                can = (dmin2 * L - done >= GB) & (issued == done)

                @pl.when(can)
                def _():
                    _issue(done)
                issued = jnp.where(can, done + GB, issued)
                return (cntv, done, issued)
            cntv, done, issued = lax.fori_loop(
                0, NCH, _chunk,
                (jnp.zeros((L,), jnp.int32), jnp.int32(0), jnp.int32(0)))

            dmax = _lanemax(cntv)[0]
            nb = (dmax * L - done + GB - 1) // GB
            done, issued = _drain(nb, done, issued)

            pltpu.sync_copy(acc_sum.at[pl.ds(0, NPW * D)],
                            sum_h.at[pl.ds(lo * D, NPW * D)])

            def _mean(n, _):
                nv = jnp.full((L,), n, jnp.int32)
                cv = plsc.load_gather(acc_cnt, [nv])
                r = 1.0 / jnp.maximum(cv, 1.0)
                for c in range(DC):
                    off = n * D + c * L
                    acc_sum[pl.ds(off, L)] = acc_sum[pl.ds(off, L)] * r
                return 0
            lax.fori_loop(0, NPW, _mean, 0)
            pltpu.sync_copy(acc_sum.at[pl.ds(0, NPW * D)],
                            mean_h.at[pl.ds(lo * D, NPW * D)])
            pltpu.sync_copy(acc_max.at[pl.ds(0, NPW * D)],
                            max_h.at[pl.ds(lo * D, NPW * D)])

    RB1 = 1000

    def _lin_body(x_ref, w_ref, b_ref, o_ref):
        o_ref[...] = jnp.dot(x_ref[...], w_ref[...],
                             preferred_element_type=jnp.float32) + b_ref[...]

    tc_lin = pl.pallas_call(
        _lin_body,
        grid=(N // RB1,),
        in_specs=[
            pl.BlockSpec((RB1, D), lambda i: (i, 0)),
            pl.BlockSpec((D, D), lambda i: (0, 0)),
            pl.BlockSpec((1, D), lambda i: (0, 0)),
        ],
        out_specs=pl.BlockSpec((RB1, D), lambda i: (i, 0)),
        out_shape=jax.ShapeDtypeStruct((N, D), jnp.float32),
    )

    RB2 = 1000

    def _post_body(s_ref, m_ref, x_ref, xin_ref, wa_ref, ba_ref, wg_ref,
                   bg_ref, ws_ref, bs_ref, o_ref):
        s = s_ref[...]
        mn = m_ref[...]
        mx = x_ref[...]
        xb = xin_ref[...]
        mask = (jnp.sum(jnp.abs(s), axis=1) + jnp.sum(jnp.abs(mn), axis=1)
                + jnp.sum(jnp.abs(mx), axis=1)) == 0.0
        wa = wa_ref[...]
        a = (jnp.dot(s, wa[0:D], preferred_element_type=jnp.float32)
             + jnp.dot(mn, wa[D:2 * D], preferred_element_type=jnp.float32)
             + jnp.dot(mx, wa[2 * D:3 * D], preferred_element_type=jnp.float32)
             + ba_ref[...])
        a = 1.0 / (1.0 + jnp.exp(-a))
        wg = wg_ref[...]
        t = (jnp.dot(s * a[:, 0:D], wg[0:D],
                     preferred_element_type=jnp.float32)
             + jnp.dot(mn * a[:, D:2 * D], wg[D:2 * D],
                       preferred_element_type=jnp.float32)
             + jnp.dot(mx * a[:, 2 * D:3 * D], wg[2 * D:3 * D],
                       preferred_element_type=jnp.float32)
             + bg_ref[...])
        t = jnp.where(mask[:, None], 0.0, t)
        out = _leaky(t)
        mask2 = jnp.sum(jnp.abs(out), axis=1) == 0.0
        ws = ws_ref[...]
        h = (jnp.dot(out, ws[0:D], preferred_element_type=jnp.float32)
             + jnp.dot(xb, ws[D:2 * D], preferred_element_type=jnp.float32)
             + bs_ref[...])
        h = _leaky(h)
        o_ref[...] = jnp.where(mask2[:, None], xb, h)

    tc_post = pl.pallas_call(
        _post_body,
        grid=(N // RB2,),
        in_specs=[
            pl.BlockSpec((RB2, D), lambda i: (i, 0)),
            pl.BlockSpec((RB2, D), lambda i: (i, 0)),
            pl.BlockSpec((RB2, D), lambda i: (i, 0)),
            pl.BlockSpec((RB2, D), lambda i: (i, 0)),
            pl.BlockSpec((3 * D, 3 * D), lambda i: (0, 0)),
            pl.BlockSpec((1, 3 * D), lambda i: (0, 0)),
            pl.BlockSpec((3 * D, D), lambda i: (0, 0)),
            pl.BlockSpec((1, D), lambda i: (0, 0)),
            pl.BlockSpec((2 * D, D), lambda i: (0, 0)),
            pl.BlockSpec((1, D), lambda i: (0, 0)),
        ],
        out_specs=pl.BlockSpec((RB2, D), lambda i: (i, 0)),
        out_shape=jax.ShapeDtypeStruct((N, D), jnp.float32),
    )

    def run(x, edge_index, W_msg, b_msg, W_att, b_att, W_aggr, b_aggr,
            W_self, b_self):
        y = tc_lin(x, W_msg, b_msg.reshape(1, D))
        src = edge_index[0]
        dst = edge_index[1]
        pad = EP - E
        src_p = jnp.concatenate([src, jnp.zeros((pad,), jnp.int32)])
        dst_p = jnp.concatenate([dst, jnp.full((pad,), SENTINEL, jnp.int32)])
        sum_t, mean_t, max_t = sc_seg(y, src_p, dst_p)
        sum_t = sum_t.reshape(NPAD, D)
        mean_t = mean_t.reshape(NPAD, D)
        max_t = max_t.reshape(NPAD, D)
        return tc_post(sum_t[:N], mean_t[:N], max_t[:N], x,
                       W_att, b_att.reshape(1, 3 * D),
                       W_aggr, b_aggr.reshape(1, D),
                       W_self, b_self.reshape(1, D))

    return run


def kernel(x, edge_index, W_msg, b_msg, W_att, b_att, W_aggr, b_aggr,
           W_self, b_self):
    N, D = x.shape
    E = edge_index.shape[1]
    return _build(N, E, D)(x, edge_index, W_msg, b_msg, W_att, b_att,
                           W_aggr, b_aggr, W_self, b_self)
